# baseline (speedup 1.0000x reference)
import jax
import jax.numpy as jnp
from jax.experimental import pallas as pl
from jax.experimental.pallas import tpu as pltpu

_CHUNK_ROWS = 1024
_NBUF = 6
_AHEAD = 3


def _copy_body(x_ref, o_ref, bufs, rsems, wsems):
    n = x_ref.shape[0] // _CHUNK_ROWS

    def rd(i):
        return pltpu.make_async_copy(
            x_ref.at[pl.ds(i * _CHUNK_ROWS, _CHUNK_ROWS), :],
            bufs.at[i % _NBUF],
            rsems.at[i % _NBUF],
        )

    def wr(i):
        return pltpu.make_async_copy(
            bufs.at[i % _NBUF],
            o_ref.at[pl.ds(i * _CHUNK_ROWS, _CHUNK_ROWS), :],
            wsems.at[i % _NBUF],
        )

    for j in range(min(_AHEAD, n)):
        rd(j).start()
    for i in range(n):
        rd(i).wait()
        wr(i).start()
        nxt = i + _AHEAD
        if nxt < n:
            if nxt >= _NBUF:
                wr(nxt - _NBUF).wait()
            rd(nxt).start()
    for i in range(max(0, n - _NBUF), n):
        wr(i).wait()


def kernel(x):
    b, s, d = x.shape
    rows = b * s
    xr = x.reshape(rows, d)
    out = pl.pallas_call(
        _copy_body,
        out_shape=jax.ShapeDtypeStruct(xr.shape, xr.dtype),
        in_specs=[pl.BlockSpec(memory_space=pltpu.HBM)],
        out_specs=pl.BlockSpec(memory_space=pltpu.HBM),
        scratch_shapes=[
            pltpu.VMEM((_NBUF, _CHUNK_ROWS, d), jnp.float32),
            pltpu.SemaphoreType.DMA((_NBUF,)),
            pltpu.SemaphoreType.DMA((_NBUF,)),
        ],
        compiler_params=pltpu.CompilerParams(
            vmem_limit_bytes=64 * 1024 * 1024,
        ),
    )(xr)
    return out.reshape(b, s, d)


# speedup vs baseline: 1.0052x; 1.0052x over previous
import jax
import jax.numpy as jnp
from jax.experimental import pallas as pl
from jax.experimental.pallas import tpu as pltpu

_BLOCK_ROWS = 2016


def _copy_body(x_ref, o_ref):
    o_ref[...] = x_ref[...]


def kernel(x):
    b, s, d = x.shape
    rows = b * s
    xr = x.reshape(rows, d)
    grid = pl.cdiv(rows, _BLOCK_ROWS)
    out = pl.pallas_call(
        _copy_body,
        out_shape=jax.ShapeDtypeStruct(xr.shape, xr.dtype),
        grid=(grid,),
        in_specs=[pl.BlockSpec((_BLOCK_ROWS, d), lambda i: (i, 0))],
        out_specs=pl.BlockSpec((_BLOCK_ROWS, d), lambda i: (i, 0)),
        compiler_params=pltpu.CompilerParams(
            dimension_semantics=("arbitrary",),
            vmem_limit_bytes=64 * 1024 * 1024,
        ),
    )(xr)
    return out.reshape(b, s, d)
